# Initial kernel scaffold; baseline (speedup 1.0000x reference)
import jax
import jax.numpy as jnp
from jax.experimental import pallas as pl


def kernel(memory, memory_valid, features, indices):
    raise NotImplementedError("write your pallas kernel here")


# speedup vs baseline: 1.9831x; 1.9831x over previous
import functools

import jax
import jax.numpy as jnp
from jax import lax
from jax.experimental import pallas as pl
from jax.experimental.pallas import tpu as pltpu
from jax.experimental.pallas import tpu_sc as plsc

MEM_ROWS = 1_000_000
FDIM = 32
BATCH = 16384

NUM_CORES = 2
NUM_SUBCORES = 16
LANES = 16
NW = NUM_CORES * NUM_SUBCORES
ROWS_PER_W = MEM_ROWS // NW
NVREG = BATCH // LANES
CAP = BATCH + LANES


def _scatter_body(mem_ref, valid_ref, feat_hbm, idx_hbm,
                  idx_all, sel_idx, sel_pos, stamp, feat_buf, ones_buf,
                  sem_in, sem_g, sem_s, sem_v):
    wid = lax.axis_index("s") * NUM_CORES + lax.axis_index("c")
    lo = wid * ROWS_PER_W

    pltpu.async_copy(idx_hbm, idx_all, sem_in).wait()

    ones_buf[...] = jnp.ones((LANES,), jnp.int32)
    lane = lax.iota(jnp.int32, LANES)

    def comp_body(i, off):
        v = idx_all[pl.ds(i * LANES, LANES)]
        pos = lane + i * LANES
        m = (v >= lo) & (v < lo + ROWS_PER_W)
        mi = m.astype(jnp.int32)
        dst = off + plsc.cumsum(mi) - mi
        plsc.store_scatter(sel_idx, [dst], v, mask=m)
        plsc.store_scatter(sel_pos, [dst], pos, mask=m)
        return off + jnp.sum(mi)

    count = lax.fori_loop(0, NVREG, comp_body, jnp.int32(0))

    @pl.when(count > 0)
    def _():
        def stamp_body(i, carry):
            linear = lane + i * LANES
            vl = linear < count
            v = sel_idx[pl.ds(i * LANES, LANES)] - lo
            for s in range(LANES):
                plsc.store_scatter(stamp, [v], linear, mask=vl & (lane == s))
            return carry

        ngrp_in = (count + LANES - 1) // LANES
        lax.fori_loop(0, ngrp_in, stamp_body, jnp.int32(0))

        def keep_body(i, foff):
            linear = lane + i * LANES
            valid_lane = linear < count
            v = sel_idx[pl.ds(i * LANES, LANES)]
            p = sel_pos[pl.ds(i * LANES, LANES)]
            g = plsc.load_gather(stamp, [v - lo], mask=valid_lane)
            keep = valid_lane & (g == linear)
            ki = keep.astype(jnp.int32)
            dst = foff + plsc.cumsum(ki) - ki
            plsc.store_scatter(sel_idx, [dst], v, mask=keep)
            plsc.store_scatter(sel_pos, [dst], p, mask=keep)
            return foff + jnp.sum(ki)

        fcount = lax.fori_loop(0, ngrp_in, keep_body, jnp.int32(0))

        last_i = sel_idx[pl.ds(fcount - 1, LANES)][0]
        last_p = sel_pos[pl.ds(fcount - 1, LANES)][0]
        sel_idx[pl.ds(fcount, LANES)] = jnp.full((LANES,), last_i, jnp.int32)
        sel_pos[pl.ds(fcount, LANES)] = jnp.full((LANES,), last_p, jnp.int32)

        def grp_body(gi, carry):
            pv = sel_pos[pl.ds(gi * LANES, LANES)]
            iv = sel_idx[pl.ds(gi * LANES, LANES)]
            pltpu.async_copy(feat_hbm.at[pv], feat_buf, sem_g).wait()
            pltpu.async_copy(feat_buf, mem_ref.at[iv], sem_s).wait()
            pltpu.async_copy(ones_buf, valid_ref.at[iv], sem_v).wait()
            return carry

        ngrp = (fcount + LANES - 1) // LANES
        lax.fori_loop(0, ngrp, grp_body, jnp.int32(0))


@functools.cache
def _sc_scatter():
    return pl.kernel(
        _scatter_body,
        out_type=(),
        mesh=plsc.VectorSubcoreMesh(
            core_axis_name="c", subcore_axis_name="s",
            num_cores=NUM_CORES, num_subcores=NUM_SUBCORES),
        compiler_params=pltpu.CompilerParams(
            needs_layout_passes=False, use_tc_tiling_on_sc=False),
        scratch_types=[
            pltpu.VMEM((BATCH,), jnp.int32),
            pltpu.VMEM((CAP,), jnp.int32),
            pltpu.VMEM((CAP,), jnp.int32),
            pltpu.VMEM((ROWS_PER_W,), jnp.int32),
            pltpu.VMEM((LANES, FDIM), jnp.float32),
            pltpu.VMEM((LANES,), jnp.int32),
            pltpu.SemaphoreType.DMA,
            pltpu.SemaphoreType.DMA,
            pltpu.SemaphoreType.DMA,
            pltpu.SemaphoreType.DMA,
        ],
    )


def kernel(memory, memory_valid, features, indices):
    valid32 = memory_valid.astype(jnp.int32)
    mem_ref = jax.new_ref(memory)
    val_ref = jax.new_ref(valid32)
    _sc_scatter()(mem_ref, val_ref, features, indices)
    return mem_ref[...], (val_ref[...] != 0)


# speedup vs baseline: 2.3132x; 1.1664x over previous
import functools

import jax
import jax.numpy as jnp
from jax import lax
from jax.experimental import pallas as pl
from jax.experimental.pallas import tpu as pltpu
from jax.experimental.pallas import tpu_sc as plsc

MEM_ROWS = 1_000_000
FDIM = 32
FPAD = 128
BATCH = 16384

NUM_CORES = 2
NUM_SUBCORES = 16
LANES = 16
NW = NUM_CORES * NUM_SUBCORES
ROWS_BASE = 31248
ROWS_LAST = MEM_ROWS - (NW - 1) * ROWS_BASE
NVREG = BATCH // LANES
CAP = BATCH + LANES
VCH = 2048


def _scatter_body(mem_ref, valid_hbm, feat_hbm, idx_hbm, valid_out,
                  idx_all, sel_idx, sel_pos, stamp, feat_buf, vbuf,
                  sem_in, sem_g, sem_s, sem_v):
    wid = lax.axis_index("s") * NUM_CORES + lax.axis_index("c")
    lo = wid * ROWS_BASE
    rows_w = jnp.where(wid == NW - 1, ROWS_LAST, ROWS_BASE)

    cp_idx = pltpu.async_copy(idx_hbm, idx_all, sem_in)

    def clr_body(i, carry):
        stamp[pl.ds(i * LANES, LANES)] = jnp.full((LANES,), -1, jnp.int32)
        return carry

    lax.fori_loop(0, ROWS_LAST // LANES, clr_body, jnp.int32(0))
    cp_idx.wait()

    lane = lax.iota(jnp.int32, LANES)

    def comp_body(i, off):
        v = idx_all[pl.ds(i * LANES, LANES)]
        pos = lane + i * LANES
        m = (v >= lo) & (v < lo + rows_w)
        mi = m.astype(jnp.int32)
        dst = off + plsc.cumsum(mi) - mi
        plsc.store_scatter(sel_idx, [dst], v, mask=m)
        plsc.store_scatter(sel_pos, [dst], pos, mask=m)
        return off + jnp.sum(mi)

    count = lax.fori_loop(0, NVREG, comp_body, jnp.int32(0))

    @pl.when(count > 0)
    def _():
        def stamp_body(i, carry):
            linear = lane + i * LANES
            vl = linear < count
            v = sel_idx[pl.ds(i * LANES, LANES)] - lo
            for s in range(LANES):
                plsc.store_scatter(stamp, [v], linear, mask=vl & (lane == s))
            return carry

        ngrp_in = (count + LANES - 1) // LANES
        lax.fori_loop(0, ngrp_in, stamp_body, jnp.int32(0))

        def keep_body(i, foff):
            linear = lane + i * LANES
            valid_lane = linear < count
            v = sel_idx[pl.ds(i * LANES, LANES)]
            p = sel_pos[pl.ds(i * LANES, LANES)]
            g = plsc.load_gather(stamp, [v - lo], mask=valid_lane)
            keep = valid_lane & (g == linear)
            ki = keep.astype(jnp.int32)
            dst = foff + plsc.cumsum(ki) - ki
            plsc.store_scatter(sel_idx, [dst], v, mask=keep)
            plsc.store_scatter(sel_pos, [dst], p, mask=keep)
            return foff + jnp.sum(ki)

        fcount = lax.fori_loop(0, ngrp_in, keep_body, jnp.int32(0))

        pad_pos = jnp.full((LANES,), fcount - 1, jnp.int32)
        last_i_v = plsc.load_gather(sel_idx, [pad_pos])
        last_p_v = plsc.load_gather(sel_pos, [pad_pos])
        base16 = (fcount // LANES) * LANES
        tmask = (base16 + lane) >= fcount
        plsc.store_scatter(sel_idx, [base16 + lane], last_i_v, mask=tmask)
        plsc.store_scatter(sel_pos, [base16 + lane], last_p_v, mask=tmask)

        def grp_body(gi, carry):
            pv = sel_pos[pl.ds(gi * LANES, LANES)]
            iv = sel_idx[pl.ds(gi * LANES, LANES)]
            for l in range(LANES):
                pltpu.async_copy(feat_hbm.at[pl.ds(pv[l], 1), :],
                                 mem_ref.at[pl.ds(iv[l], 1), :], sem_s)
            for l in range(LANES):
                pltpu.make_async_copy(feat_hbm.at[pl.ds(pv[l], 1), :],
                                      mem_ref.at[pl.ds(iv[l], 1), :],
                                      sem_s).wait()
            return carry

        ngrp = (fcount + LANES - 1) // LANES
        lax.fori_loop(0, ngrp, grp_body, jnp.int32(0))

    def vch_body(ci, off):
        base = pl.multiple_of(lo + off, 8)
        pltpu.async_copy(valid_hbm.at[pl.ds(base, VCH)], vbuf, sem_v).wait()

        def or_body(k, carry):
            o = vbuf[pl.ds(k * LANES, LANES)]
            st = stamp[pl.ds(pl.multiple_of(off + k * LANES, 8), LANES)]
            vbuf[pl.ds(k * LANES, LANES)] = o | (st >= 0).astype(jnp.int32)
            return carry

        lax.fori_loop(0, VCH // LANES, or_body, jnp.int32(0))
        pltpu.async_copy(vbuf, valid_out.at[pl.ds(base, VCH)], sem_v).wait()
        return off + VCH

    lax.fori_loop(0, rows_w // VCH, vch_body, jnp.int32(0))
    vch_body(jnp.int32(0), rows_w - VCH)


@functools.cache
def _sc_scatter():
    return pl.kernel(
        _scatter_body,
        out_type=jax.ShapeDtypeStruct((MEM_ROWS,), jnp.int32),
        mesh=plsc.VectorSubcoreMesh(
            core_axis_name="c", subcore_axis_name="s",
            num_cores=NUM_CORES, num_subcores=NUM_SUBCORES),
        compiler_params=pltpu.CompilerParams(needs_layout_passes=False),
        scratch_types=[
            pltpu.VMEM((BATCH,), jnp.int32),
            pltpu.VMEM((CAP,), jnp.int32),
            pltpu.VMEM((CAP,), jnp.int32),
            pltpu.VMEM((ROWS_LAST,), jnp.int32),
            pltpu.VMEM((LANES, FDIM), jnp.float32),
            pltpu.VMEM((VCH,), jnp.int32),
            pltpu.SemaphoreType.DMA,
            pltpu.SemaphoreType.DMA,
            pltpu.SemaphoreType.DMA,
            pltpu.SemaphoreType.DMA,
        ],
    )


def kernel(memory, memory_valid, features, indices):
    valid32 = memory_valid.astype(jnp.int32)
    mem_ref = jax.new_ref(memory)
    valid_new = _sc_scatter()(mem_ref, valid32, features, indices)
    return mem_ref[...], (valid_new != 0)
